# baseline (speedup 1.0000x reference)
import functools

import jax
import jax.numpy as jnp
import numpy as np
from jax import lax
from jax.experimental import pallas as pl
from jax.experimental.pallas import tpu as pltpu
from jax.experimental.pallas import tpu_sc as plsc

ROWS = 128
COLS = 32768
KTOP = 64
NC = 2
NS = 16
L = 16
NW = NC * NS
RPW = ROWS // NW
NVEC = COLS // L
NB = 256
SSTRIDE = 16
SMIN = 16

_MESH = plsc.VectorSubcoreMesh(
    core_axis_name="c", subcore_axis_name="s", num_cores=NC, num_subcores=NS
)

_MASK31 = np.int32(0x7FFFFFFF)


def _keyize(u):
    return u ^ (lax.shift_right_arithmetic(u, 31) & _MASK31)


def _xlane(v, perm):
    return v.at[perm].get(mode="promise_in_bounds")


def _clean16(v, iota):
    for s in (8, 4, 2, 1):
        p = _xlane(v, iota ^ s)
        take_min = (iota & s) == 0
        v = jnp.where(take_min, jnp.minimum(v, p), jnp.maximum(v, p))
    return v


def _merge16(a, b, iota):
    br = lax.rev(b, (0,))
    lo = jnp.minimum(a, br)
    hi = jnp.maximum(a, br)
    return _clean16(lo, iota), _clean16(hi, iota)


def _sort64(d0, d1, d2, d3, iota):
    s0 = jnp.sort(d0)
    s1 = jnp.sort(d1)
    s2 = jnp.sort(d2)
    s3 = jnp.sort(d3)
    a0, a1 = _merge16(s0, s1, iota)
    b0, b1 = _merge16(s2, s3, iota)
    rb1 = lax.rev(b1, (0,))
    rb0 = lax.rev(b0, (0,))
    l0 = jnp.minimum(a0, rb1)
    l1 = jnp.minimum(a1, rb0)
    h0 = jnp.maximum(a0, rb1)
    h1 = jnp.maximum(a1, rb0)
    e0 = _clean16(jnp.minimum(l0, l1), iota)
    e1 = _clean16(jnp.maximum(l0, l1), iota)
    e2 = _clean16(jnp.minimum(h0, h1), iota)
    e3 = _clean16(jnp.maximum(h0, h1), iota)
    return e0, e1, e2, e3


def _body(x_hbm, out_hbm, xa_v, xb_v, cand_v, hist_v, def_v, out_v, sa, sb, so):
    wid = lax.axis_index("s") * NC + lax.axis_index("c")
    iota = lax.iota(jnp.int32, L)
    zeros16 = jnp.zeros((L,), jnp.int32)
    ones16 = jnp.ones((L,), jnp.int32)
    last16 = jnp.full((L,), L - 1, jnp.int32)

    def zero_hist():
        for i in range(NB // L):
            hist_v[pl.ds(i * L, L)] = zeros16

    def find_threshold(k):
        @plsc.parallel_loop(0, NB // L, step=1, unroll=4, carry=zeros16)
        def bsums(i, acc):
            c = plsc.cumsum(hist_v[pl.ds(i * L, L)])
            return acc + jnp.where(iota == i, _xlane(c, last16), 0)
        br = lax.rev(bsums, (0,))
        cb = plsc.cumsum(br)
        tb = plsc.all_reduce_ffs(cb >= k)
        fb = (NB // L - 1) - tb
        above_blk = _xlane(cb, tb) - _xlane(br, tb)
        h = plsc.load_gather(hist_v, [fb * L + iota])
        hr = lax.rev(h, (0,))
        c = plsc.cumsum(hr)
        crossed = (above_blk + c) >= k
        ts = plsc.all_reduce_ffs(crossed)
        bst = fb * L + (L - 1) - ts
        above = above_blk + _xlane(c, ts) - _xlane(hr, ts)
        return bst, above

    def refine_level(bucket_fn, count, k, cur_d):
        zero_hist()
        nv = (count + (L - 1)) // L

        @plsc.parallel_loop(0, nv, step=1, unroll=2)
        def hst(i):
            sk = cand_v[pl.ds(i * L, L)]
            valid = (i * L + iota) < count
            plsc.addupdate_scatter(hist_v, [bucket_fn(sk)], ones16, mask=valid)

        bst, above = find_threshold(k)

        @plsc.parallel_loop(0, nv, step=1, unroll=2, carry=(cur_d - 1, zeros16 - 1))
        def flt(i, carry):
            cd, cc = carry
            sk = cand_v[pl.ds(i * L, L)]
            valid = (i * L + iota) < count
            b = bucket_fn(sk)
            gt = jnp.logical_and(valid, b > bst)
            eq = jnp.logical_and(valid, b == bst)
            pg = plsc.cumsum(ones16, mask=gt)
            plsc.store_scatter(def_v, [cd + pg], sk, mask=gt)
            pe = plsc.cumsum(ones16, mask=eq)
            plsc.store_scatter(cand_v, [cc + pe], sk, mask=eq)
            return (
                cd + plsc.all_reduce_population_count(gt),
                cc + plsc.all_reduce_population_count(eq),
            )
        cur_d, cur_c = flt
        return cur_d + 1, jnp.max(cur_c) + 1, bst, above

    def bucket_b1(sk):
        return lax.shift_right_arithmetic(sk, 24) + 128

    def bucket_b2(sk):
        return lax.shift_right_arithmetic(sk, 16) & jnp.int32(0xFF)

    def process_row(x_v, r, j):
        zero_hist()

        @plsc.parallel_loop(0, NVEC // SSTRIDE, step=1, unroll=4)
        def samp(i):
            x = x_v[pl.ds(i * (SSTRIDE * L), L)]
            sk = _keyize(lax.bitcast_convert_type(x, jnp.int32))
            plsc.addupdate_scatter(hist_v, [bucket_b1(sk)], ones16)
        bst_s, above_s = find_threshold(jnp.full((L,), SMIN, jnp.int32))
        zero_hist()

        @plsc.parallel_loop(0, NVEC // SSTRIDE, step=1, unroll=4)
        def samp2(i):
            x = x_v[pl.ds(i * (SSTRIDE * L), L)]
            sk = _keyize(lax.bitcast_convert_type(x, jnp.int32))
            m = bucket_b1(sk) == bst_s
            plsc.addupdate_scatter(hist_v, [bucket_b2(sk)], ones16, mask=m)
        bst2_s, _ = find_threshold(jnp.int32(SMIN) - above_s)
        tk = lax.shift_left(bst_s - 128, 24) | lax.shift_left(bst2_s, 16)
        t_low = lax.bitcast_convert_type(
            tk ^ (lax.shift_right_arithmetic(tk, 31) & _MASK31), jnp.float32)

        def compact_pass(t):
            @plsc.parallel_loop(0, NVEC, step=1, unroll=16, carry=zeros16 - 1)
            def pb(i, cc):
                x = x_v[pl.ds(i * L, L)]
                hot = x >= t
                p = plsc.cumsum(ones16, mask=hot)
                plsc.store_scatter(
                    cand_v, [cc + p],
                    lax.bitcast_convert_type(x, jnp.int32), mask=hot)
                return cc + plsc.all_reduce_population_count(hot)
            return jnp.max(pb) + 1

        def trip(s, carry):
            count, t = carry
            need = jnp.logical_or(s == 0, count < KTOP)
            count = lax.cond(need, lambda: compact_pass(t), lambda: count)
            return (count, jnp.full((L,), -jnp.inf, jnp.float32))
        count, _ = lax.fori_loop(0, 2, trip, (jnp.int32(0), t_low))

        @plsc.parallel_loop(0, (count + (L - 1)) // L, step=1, unroll=4)
        def kz(i):
            u = cand_v[pl.ds(i * L, L)]
            cand_v[pl.ds(i * L, L)] = _keyize(u)

        def lvl_body(lvl, carry):
            cur_d, cnt, k, tacc = carry
            shift = 24 - 8 * lvl
            bias = jnp.where(lvl == 0, jnp.int32(0x80), jnp.int32(0))

            def bucket_fn(sk):
                return (
                    lax.shift_right_arithmetic(sk, shift) & jnp.int32(0xFF)
                ) ^ bias
            cur_d, cnt2, bst, above = refine_level(bucket_fn, cnt, k, cur_d)
            tacc = tacc | lax.shift_left(bst ^ bias, shift)
            return (cur_d, cnt2, k - above, tacc)
        cur_d, _cn, k4, t_key = lax.fori_loop(
            0, 4, lvl_body,
            (zeros16, count, jnp.full((L,), KTOP, jnp.int32), zeros16))
        for t in range(4):
            m = (t * L + iota) < k4
            idx = cur_d + t * L + iota
            plsc.store_scatter(def_v, [idx], t_key, mask=m)

        d0 = def_v[pl.ds(0, L)]
        d1 = def_v[pl.ds(L, L)]
        d2 = def_v[pl.ds(2 * L, L)]
        d3 = def_v[pl.ds(3 * L, L)]
        e0, e1, e2, e3 = _sort64(d0, d1, d2, d3, iota)
        for t, e in enumerate((e3, e2, e1, e0)):
            w = lax.rev(e, (0,))
            u = w ^ (lax.shift_right_arithmetic(w, 31) & _MASK31)
            out_v[j, pl.ds(t * L, L)] = lax.bitcast_convert_type(
                u, jnp.float32)
        pltpu.async_copy(out_v.at[j], out_hbm.at[r], so)

    r0 = wid * RPW
    nhalf = RPW // 2
    pltpu.async_copy(x_hbm.at[r0], xa_v, sa)
    pltpu.async_copy(x_hbm.at[r0 + 1], xb_v, sb)

    def rows(jo, c):
        r = r0 + 2 * jo
        pltpu.make_async_copy(x_hbm.at[r], xa_v, sa).wait()
        process_row(xa_v, r, 2 * jo)

        @pl.when(jo < nhalf - 1)
        def _():
            pltpu.async_copy(x_hbm.at[r + 2], xa_v, sa)
        pltpu.make_async_copy(x_hbm.at[r + 1], xb_v, sb).wait()
        process_row(xb_v, r + 1, 2 * jo + 1)

        @pl.when(jo < nhalf - 1)
        def _():
            pltpu.async_copy(x_hbm.at[r + 3], xb_v, sb)
        return c
    lax.fori_loop(0, nhalf, rows, 0)
    for _ in range(RPW):
        pltpu.make_async_copy(out_v.at[0], out_hbm.at[r0], so).wait()


_topk_sc = functools.partial(
    pl.kernel,
    out_type=jax.ShapeDtypeStruct((ROWS, KTOP), jnp.float32),
    mesh=_MESH,
    compiler_params=pltpu.CompilerParams(needs_layout_passes=False),
    scratch_types=[
        pltpu.VMEM((COLS,), jnp.float32),
        pltpu.VMEM((COLS,), jnp.float32),
        pltpu.VMEM((COLS,), jnp.int32),
        pltpu.VMEM((NB,), jnp.int32),
        pltpu.VMEM((2 * KTOP,), jnp.int32),
        pltpu.VMEM((RPW, KTOP), jnp.float32),
        pltpu.SemaphoreType.DMA,
        pltpu.SemaphoreType.DMA,
        pltpu.SemaphoreType.DMA,
    ],
)(_body)


def kernel(input):
    return _topk_sc(input)


# speedup vs baseline: 1.0020x; 1.0020x over previous
import functools

import jax
import jax.numpy as jnp
import numpy as np
from jax import lax
from jax.experimental import pallas as pl
from jax.experimental.pallas import tpu as pltpu
from jax.experimental.pallas import tpu_sc as plsc

ROWS = 128
COLS = 32768
KTOP = 64
NC = 2
NS = 16
L = 16
NW = NC * NS
RPW = ROWS // NW
NVEC = COLS // L
NB = 256
SSTRIDE = 16
SMIN = 16

_MASK31 = np.int32(0x7FFFFFFF)


def _keyize(u):
    return u ^ (lax.shift_right_arithmetic(u, 31) & _MASK31)


def _xlane(v, perm):
    return v.at[perm].get(mode="promise_in_bounds")


def _clean16(v, iota):
    for s in (8, 4, 2, 1):
        p = _xlane(v, iota ^ s)
        take_min = (iota & s) == 0
        v = jnp.where(take_min, jnp.minimum(v, p), jnp.maximum(v, p))
    return v


def _merge16(a, b, iota):
    br = lax.rev(b, (0,))
    lo = jnp.minimum(a, br)
    hi = jnp.maximum(a, br)
    return _clean16(lo, iota), _clean16(hi, iota)


def _sort64(d0, d1, d2, d3, iota):
    s0 = jnp.sort(d0)
    s1 = jnp.sort(d1)
    s2 = jnp.sort(d2)
    s3 = jnp.sort(d3)
    a0, a1 = _merge16(s0, s1, iota)
    b0, b1 = _merge16(s2, s3, iota)
    rb1 = lax.rev(b1, (0,))
    rb0 = lax.rev(b0, (0,))
    l0 = jnp.minimum(a0, rb1)
    l1 = jnp.minimum(a1, rb0)
    h0 = jnp.maximum(a0, rb1)
    h1 = jnp.maximum(a1, rb0)
    e0 = _clean16(jnp.minimum(l0, l1), iota)
    e1 = _clean16(jnp.maximum(l0, l1), iota)
    e2 = _clean16(jnp.minimum(h0, h1), iota)
    e3 = _clean16(jnp.maximum(h0, h1), iota)
    return e0, e1, e2, e3


def _body(x_hbm, out_hbm, xa_v, xb_v, cand_v, hist_v, def_v, out_v, sa, sb, so):
    wid = lax.axis_index("s") * NC + lax.axis_index("c")
    iota = lax.iota(jnp.int32, L)
    zeros16 = jnp.zeros((L,), jnp.int32)
    ones16 = jnp.ones((L,), jnp.int32)
    last16 = jnp.full((L,), L - 1, jnp.int32)

    def zero_hist():
        for i in range(NB // L):
            hist_v[pl.ds(i * L, L)] = zeros16

    def find_threshold(k):
        @plsc.parallel_loop(0, NB // L, step=1, unroll=4, carry=zeros16)
        def bsums(i, acc):
            c = plsc.cumsum(hist_v[pl.ds(i * L, L)])
            return acc + jnp.where(iota == i, _xlane(c, last16), 0)
        br = lax.rev(bsums, (0,))
        cb = plsc.cumsum(br)
        tb = plsc.all_reduce_ffs(cb >= k)
        fb = (NB // L - 1) - tb
        above_blk = _xlane(cb, tb) - _xlane(br, tb)
        h = plsc.load_gather(hist_v, [fb * L + iota])
        hr = lax.rev(h, (0,))
        c = plsc.cumsum(hr)
        crossed = (above_blk + c) >= k
        ts = plsc.all_reduce_ffs(crossed)
        bst = fb * L + (L - 1) - ts
        above = above_blk + _xlane(c, ts) - _xlane(hr, ts)
        return bst, above

    def refine_level(bucket_fn, count, k, cur_d):
        zero_hist()
        nv = (count + (L - 1)) // L

        @plsc.parallel_loop(0, nv, step=1, unroll=2)
        def hst(i):
            sk = cand_v[pl.ds(i * L, L)]
            valid = (i * L + iota) < count
            plsc.addupdate_scatter(hist_v, [bucket_fn(sk)], ones16, mask=valid)

        bst, above = find_threshold(k)

        @plsc.parallel_loop(0, nv, step=1, unroll=2, carry=(cur_d - 1, zeros16 - 1))
        def flt(i, carry):
            cd, cc = carry
            sk = cand_v[pl.ds(i * L, L)]
            valid = (i * L + iota) < count
            b = bucket_fn(sk)
            gt = jnp.logical_and(valid, b > bst)
            eq = jnp.logical_and(valid, b == bst)
            pg = plsc.cumsum(ones16, mask=gt)
            plsc.store_scatter(def_v, [cd + pg], sk, mask=gt)
            pe = plsc.cumsum(ones16, mask=eq)
            plsc.store_scatter(cand_v, [cc + pe], sk, mask=eq)
            return (
                cd + plsc.all_reduce_population_count(gt),
                cc + plsc.all_reduce_population_count(eq),
            )
        cur_d, cur_c = flt
        return cur_d + 1, jnp.max(cur_c) + 1, bst, above

    def bucket_b1(sk):
        return lax.shift_right_arithmetic(sk, 24) + 128

    def bucket_b2(sk):
        return lax.shift_right_arithmetic(sk, 16) & jnp.int32(0xFF)

    def process_row(x_v, r, j):
        zero_hist()

        @plsc.parallel_loop(0, NVEC // SSTRIDE, step=1, unroll=4)
        def samp(i):
            x = x_v[pl.ds(i * (SSTRIDE * L), L)]
            sk = _keyize(lax.bitcast_convert_type(x, jnp.int32))
            plsc.addupdate_scatter(hist_v, [bucket_b1(sk)], ones16)
        bst_s, above_s = find_threshold(jnp.full((L,), SMIN, jnp.int32))
        zero_hist()

        @plsc.parallel_loop(0, NVEC // SSTRIDE, step=1, unroll=4)
        def samp2(i):
            x = x_v[pl.ds(i * (SSTRIDE * L), L)]
            sk = _keyize(lax.bitcast_convert_type(x, jnp.int32))
            m = bucket_b1(sk) == bst_s
            plsc.addupdate_scatter(hist_v, [bucket_b2(sk)], ones16, mask=m)
        bst2_s, _ = find_threshold(jnp.int32(SMIN) - above_s)
        tk = lax.shift_left(bst_s - 128, 24) | lax.shift_left(bst2_s, 16)
        t_low = lax.bitcast_convert_type(
            tk ^ (lax.shift_right_arithmetic(tk, 31) & _MASK31), jnp.float32)

        def compact_pass(t):
            @plsc.parallel_loop(0, NVEC, step=1, unroll=16, carry=zeros16 - 1)
            def pb(i, cc):
                x = x_v[pl.ds(i * L, L)]
                hot = x >= t
                p = plsc.cumsum(ones16, mask=hot)
                plsc.store_scatter(
                    cand_v, [cc + p],
                    lax.bitcast_convert_type(x, jnp.int32), mask=hot)
                return cc + plsc.all_reduce_population_count(hot)
            return jnp.max(pb) + 1

        def trip(s, carry):
            count, t = carry
            need = jnp.logical_or(s == 0, count < KTOP)
            count = lax.cond(need, lambda: compact_pass(t), lambda: count)
            return (count, jnp.full((L,), -jnp.inf, jnp.float32))
        count, _ = lax.fori_loop(0, 2, trip, (jnp.int32(0), t_low))

        @plsc.parallel_loop(0, (count + (L - 1)) // L, step=1, unroll=4)
        def kz(i):
            u = cand_v[pl.ds(i * L, L)]
            cand_v[pl.ds(i * L, L)] = _keyize(u)

        def lvl_body(lvl, carry):
            cur_d, cnt, k, tacc = carry
            shift = 24 - 8 * lvl
            bias = jnp.where(lvl == 0, jnp.int32(0x80), jnp.int32(0))

            def bucket_fn(sk):
                return (
                    lax.shift_right_arithmetic(sk, shift) & jnp.int32(0xFF)
                ) ^ bias
            cur_d, cnt2, bst, above = refine_level(bucket_fn, cnt, k, cur_d)
            tacc = tacc | lax.shift_left(bst ^ bias, shift)
            return (cur_d, cnt2, k - above, tacc)
        cur_d, _cn, k4, t_key = lax.fori_loop(
            0, 4, lvl_body,
            (zeros16, count, jnp.full((L,), KTOP, jnp.int32), zeros16))
        for t in range(4):
            m = (t * L + iota) < k4
            idx = cur_d + t * L + iota
            plsc.store_scatter(def_v, [idx], t_key, mask=m)

        d0 = def_v[pl.ds(0, L)]
        d1 = def_v[pl.ds(L, L)]
        d2 = def_v[pl.ds(2 * L, L)]
        d3 = def_v[pl.ds(3 * L, L)]
        e0, e1, e2, e3 = _sort64(d0, d1, d2, d3, iota)
        for t, e in enumerate((e3, e2, e1, e0)):
            w = lax.rev(e, (0,))
            u = w ^ (lax.shift_right_arithmetic(w, 31) & _MASK31)
            out_v[j, pl.ds(t * L, L)] = lax.bitcast_convert_type(
                u, jnp.float32)
        pltpu.async_copy(out_v.at[j], out_hbm.at[r], so)

    r0 = wid * RPW
    nhalf = RPW // 2
    pltpu.async_copy(x_hbm.at[r0], xa_v, sa)
    pltpu.async_copy(x_hbm.at[r0 + 1], xb_v, sb)

    def rows(jo, c):
        r = r0 + 2 * jo
        pltpu.make_async_copy(x_hbm.at[r], xa_v, sa).wait()
        process_row(xa_v, r, 2 * jo)

        @pl.when(jo < nhalf - 1)
        def _():
            pltpu.async_copy(x_hbm.at[r + 2], xa_v, sa)
        pltpu.make_async_copy(x_hbm.at[r + 1], xb_v, sb).wait()
        process_row(xb_v, r + 1, 2 * jo + 1)

        @pl.when(jo < nhalf - 1)
        def _():
            pltpu.async_copy(x_hbm.at[r + 3], xb_v, sb)
        return c
    lax.fori_loop(0, nhalf, rows, 0)
    for _ in range(RPW):
        pltpu.make_async_copy(out_v.at[0], out_hbm.at[r0], so).wait()


_TOPK_SC = []


def _build():
    mesh = plsc.VectorSubcoreMesh(
        core_axis_name="c", subcore_axis_name="s",
        num_cores=NC, num_subcores=NS,
    )
    return functools.partial(
        pl.kernel,
        out_type=jax.ShapeDtypeStruct((ROWS, KTOP), jnp.float32),
        mesh=mesh,
        compiler_params=pltpu.CompilerParams(needs_layout_passes=False),
        scratch_types=[
            pltpu.VMEM((COLS,), jnp.float32),
            pltpu.VMEM((COLS,), jnp.float32),
            pltpu.VMEM((COLS,), jnp.int32),
            pltpu.VMEM((NB,), jnp.int32),
            pltpu.VMEM((2 * KTOP,), jnp.int32),
            pltpu.VMEM((RPW, KTOP), jnp.float32),
            pltpu.SemaphoreType.DMA,
            pltpu.SemaphoreType.DMA,
            pltpu.SemaphoreType.DMA,
        ],
    )(_body)


def kernel(input):
    if not _TOPK_SC:
        _TOPK_SC.append(_build())
    return _TOPK_SC[0](input)
